# baseline (speedup 1.0000x reference)
import functools

import jax
import jax.numpy as jnp
from jax import lax
from jax.experimental import pallas as pl
from jax.experimental.pallas import tpu as pltpu
from jax.experimental.pallas import tpu_sc as plsc

E = 64
NC, NS = 2, 16
NW = NC * NS
LANES = 16


def _make_sc_reduce(B, L):
    upw = B // NW
    G = 2
    NG = upw // G

    mesh = plsc.VectorSubcoreMesh(core_axis_name="c", subcore_axis_name="s")

    @functools.partial(
        pl.kernel,
        out_type=jax.ShapeDtypeStruct((B, E), jnp.float32),
        mesh=mesh,
        compiler_params=pltpu.CompilerParams(use_tc_tiling_on_sc=True),
        scratch_types=[
            pltpu.VMEM((G * L, E), jnp.float32),
            pltpu.VMEM((G * L, E), jnp.float32),
            pltpu.VMEM((upw, E), jnp.float32),
            pltpu.SemaphoreType.DMA,
            pltpu.SemaphoreType.DMA,
        ],
    )
    def sc_kernel(rows_hbm, hu_hbm, rows_a, rows_b, hu_buf, sem0, sem1):
        wid = lax.axis_index("s") * NC + lax.axis_index("c")
        base = pl.multiple_of(wid * upw, upw)

        def fetch(g, buf, sem):
            off = pl.multiple_of((base + g * G) * L, 8)
            return pltpu.async_copy(rows_hbm.at[pl.ds(off, G * L)], buf, sem)

        def accum_group(g, buf):
            unroll = 4
            for gu in range(G):
                def row_body(jj, accs, _gu=gu):
                    j = _gu * L + jj * unroll
                    return tuple(
                        a + sum(buf[j + r, pl.ds(LANES * t, LANES)]
                                for r in range(1, unroll))
                        + buf[j, pl.ds(LANES * t, LANES)]
                        for t, a in enumerate(accs))

                z = jnp.zeros((LANES,), jnp.float32)
                accs = lax.fori_loop(0, L // unroll, row_body, (z, z, z, z))
                for t in range(E // LANES):
                    hu_buf[g * G + gu, pl.ds(LANES * t, LANES)] = accs[t]

        fetch(0, rows_a, sem0).wait()
        def group_body(i, carry):
            g = i * 2
            nxt = fetch(g + 1, rows_b, sem1)
            accum_group(g, rows_a)
            nxt.wait()
            nxt2 = fetch(g + 2, rows_a, sem0)
            accum_group(g + 1, rows_b)
            nxt2.wait()
            return carry

        lax.fori_loop(0, (NG - 2) // 2, group_body, 0)
        last = fetch(NG - 1, rows_b, sem1)
        accum_group(NG - 2, rows_a)
        last.wait()
        accum_group(NG - 1, rows_b)
        pltpu.sync_copy(hu_buf, hu_hbm.at[pl.ds(base, upw)])

    return sc_kernel


def _tc_reduce_body(rows_ref, out_ref):
    gu, e = out_ref.shape
    x = rows_ref[...]
    out_ref[...] = jnp.sum(x.reshape(gu, x.shape[0] // gu, e), axis=1)


def _make_tc_reduce(n_users, L, gu):
    grid = (n_users // gu,)
    return pl.pallas_call(
        _tc_reduce_body,
        grid=grid,
        in_specs=[pl.BlockSpec((gu * L, E), lambda i: (i, 0))],
        out_specs=pl.BlockSpec((gu, E), lambda i: (i, 0)),
        out_shape=jax.ShapeDtypeStruct((n_users, E), jnp.float32),
    )


def _tc_body(hist_ref, hu_ref, posh_ref, negh_ref,
             q1w_ref, q1b_ref, q2w_ref, q2b_ref,
             p1w_ref, p1b_ref, p2w_ref, p2b_ref,
             posl_ref, negl_ref, mu_ref, logvar_ref):
    cnt = jnp.sum((hist_ref[...] != 0).astype(jnp.float32), axis=1,
                  keepdims=True)
    scale = lax.rsqrt(jnp.maximum(cnt, 1.0))
    hu = hu_ref[...] * scale
    h = jnp.tanh(jnp.dot(hu, q1w_ref[...],
                         preferred_element_type=jnp.float32) + q1b_ref[...])
    h = jnp.dot(h, q2w_ref[...],
                preferred_element_type=jnp.float32) + q2b_ref[...]
    mu = h[:, :E]
    mu_ref[...] = mu
    logvar_ref[...] = h[:, E:]
    h2 = jnp.tanh(jnp.dot(mu, p1w_ref[...],
                          preferred_element_type=jnp.float32) + p1b_ref[...])
    h2 = jnp.dot(h2, p2w_ref[...],
                 preferred_element_type=jnp.float32) + p2b_ref[...]
    posl_ref[...] = jnp.sum(h2 * posh_ref[...], axis=1, keepdims=True)
    negl_ref[...] = jnp.sum(h2 * negh_ref[...], axis=1, keepdims=True)


def _make_tc_mlp(B, L, bb):
    grid = (B // bb,)
    row_spec = lambda w: pl.BlockSpec((bb, w), lambda i: (i, 0))
    rep_spec = lambda r, c: pl.BlockSpec((r, c), lambda i: (0, 0))
    return pl.pallas_call(
        _tc_body,
        grid=grid,
        in_specs=[
            row_spec(L),
            row_spec(E),
            row_spec(E),
            row_spec(E),
            rep_spec(E, E),
            rep_spec(1, E),
            rep_spec(E, 2 * E),
            rep_spec(1, 2 * E),
            rep_spec(E, E),
            rep_spec(1, E),
            rep_spec(E, E),
            rep_spec(1, E),
        ],
        out_specs=[
            row_spec(1),
            row_spec(1),
            row_spec(E),
            row_spec(E),
        ],
        out_shape=[
            jax.ShapeDtypeStruct((B, 1), jnp.float32),
            jax.ShapeDtypeStruct((B, 1), jnp.float32),
            jax.ShapeDtypeStruct((B, E), jnp.float32),
            jax.ShapeDtypeStruct((B, E), jnp.float32),
        ],
    )


def kernel(user, user_hist, user_nbrs, pos_item, neg_item, q_table, p_table,
           q1_w, q1_b, q2_w, q2_b, p1_w, p1_b, p2_w, p2_b):
    B, L = user_hist.shape
    hist = user_hist.astype(jnp.int32)
    pos = pos_item.astype(jnp.int32)
    neg = neg_item.astype(jnp.int32)

    cols = jnp.arange(E, dtype=jnp.int32)[None, :]
    pn_hi = p_table[jnp.concatenate([pos, neg])[:, None], cols]
    pos_hi, neg_hi = pn_hi[:B], pn_hi[B:]

    Ba = B // 2
    Bb = B - Ba
    rows_a = jnp.take(q_table, hist[:Ba].reshape(Ba * L), axis=0,
                      mode="clip")
    hu_a = rows_a.reshape(Ba, L, E).sum(axis=1)
    rows_b = jnp.take(q_table, hist[Ba:].reshape(Bb * L), axis=0,
                      mode="clip")
    hu_b = _make_sc_reduce(Bb, L)(rows_b)
    hu_raw = jnp.concatenate([hu_a, hu_b], axis=0)

    posl, negl, mu, logvar = _make_tc_mlp(B, L, 512)(
        hist, hu_raw, pos_hi, neg_hi,
        q1_w, q1_b.reshape(1, E), q2_w, q2_b.reshape(1, 2 * E),
        p1_w, p1_b.reshape(1, E), p2_w, p2_b.reshape(1, E))

    return (posl.reshape(B), negl.reshape(B), mu, logvar)


# speedup vs baseline: 1.0818x; 1.0818x over previous
import functools

import jax
import jax.numpy as jnp
from jax import lax
from jax.experimental import pallas as pl
from jax.experimental.pallas import tpu as pltpu
from jax.experimental.pallas import tpu_sc as plsc

E = 64
NC, NS = 2, 16
NW = NC * NS
LANES = 16


def _make_sc_reduce(B, L):
    upw = B // NW
    G = 2
    NG = upw // G

    mesh = plsc.VectorSubcoreMesh(core_axis_name="c", subcore_axis_name="s")

    @functools.partial(
        pl.kernel,
        out_type=jax.ShapeDtypeStruct((B, E), jnp.float32),
        mesh=mesh,
        compiler_params=pltpu.CompilerParams(use_tc_tiling_on_sc=True),
        scratch_types=[
            pltpu.VMEM((G * L, E), jnp.float32),
            pltpu.VMEM((G * L, E), jnp.float32),
            pltpu.VMEM((upw, E), jnp.float32),
            pltpu.SemaphoreType.DMA,
            pltpu.SemaphoreType.DMA,
        ],
    )
    def sc_kernel(rows_hbm, hu_hbm, rows_a, rows_b, hu_buf, sem0, sem1):
        wid = lax.axis_index("s") * NC + lax.axis_index("c")
        base = pl.multiple_of(wid * upw, upw)

        def fetch(g, buf, sem):
            off = pl.multiple_of((base + g * G) * L, 8)
            return pltpu.async_copy(rows_hbm.at[pl.ds(off, G * L)], buf, sem)

        def accum_group(g, buf):
            unroll = 4
            for gu in range(G):
                def row_body(jj, accs, _gu=gu):
                    j = _gu * L + jj * unroll
                    return tuple(
                        a + sum(buf[j + r, pl.ds(LANES * t, LANES)]
                                for r in range(1, unroll))
                        + buf[j, pl.ds(LANES * t, LANES)]
                        for t, a in enumerate(accs))

                z = jnp.zeros((LANES,), jnp.float32)
                accs = lax.fori_loop(0, L // unroll, row_body, (z, z, z, z))
                for t in range(E // LANES):
                    hu_buf[g * G + gu, pl.ds(LANES * t, LANES)] = accs[t]

        fetch(0, rows_a, sem0).wait()
        def group_body(i, carry):
            g = i * 2
            nxt = fetch(g + 1, rows_b, sem1)
            accum_group(g, rows_a)
            nxt.wait()
            nxt2 = fetch(g + 2, rows_a, sem0)
            accum_group(g + 1, rows_b)
            nxt2.wait()
            return carry

        lax.fori_loop(0, (NG - 2) // 2, group_body, 0)
        last = fetch(NG - 1, rows_b, sem1)
        accum_group(NG - 2, rows_a)
        last.wait()
        accum_group(NG - 1, rows_b)
        pltpu.sync_copy(hu_buf, hu_hbm.at[pl.ds(base, upw)])

    return sc_kernel


def _tc_reduce_body(rows_ref, out_ref):
    gu, e = out_ref.shape
    x = rows_ref[...]
    out_ref[...] = jnp.sum(x.reshape(gu, x.shape[0] // gu, e), axis=1)


def _make_tc_reduce(n_users, L, gu):
    grid = (n_users // gu,)
    return pl.pallas_call(
        _tc_reduce_body,
        grid=grid,
        in_specs=[pl.BlockSpec((gu * L, E), lambda i: (i, 0))],
        out_specs=pl.BlockSpec((gu, E), lambda i: (i, 0)),
        out_shape=jax.ShapeDtypeStruct((n_users, E), jnp.float32),
    )


def _tc_body(hist_ref, hu_ref, posh_ref, negh_ref,
             q1w_ref, q1b_ref, q2w_ref, q2b_ref,
             p1w_ref, p1b_ref, p2w_ref, p2b_ref,
             posl_ref, negl_ref, mu_ref, logvar_ref):
    cnt = jnp.sum((hist_ref[...] != 0).astype(jnp.float32), axis=1,
                  keepdims=True)
    scale = lax.rsqrt(jnp.maximum(cnt, 1.0))
    hu = hu_ref[...] * scale
    h = jnp.tanh(jnp.dot(hu, q1w_ref[...],
                         preferred_element_type=jnp.float32) + q1b_ref[...])
    h = jnp.dot(h, q2w_ref[...],
                preferred_element_type=jnp.float32) + q2b_ref[...]
    mu = h[:, :E]
    mu_ref[...] = mu
    logvar_ref[...] = h[:, E:]
    h2 = jnp.tanh(jnp.dot(mu, p1w_ref[...],
                          preferred_element_type=jnp.float32) + p1b_ref[...])
    h2 = jnp.dot(h2, p2w_ref[...],
                 preferred_element_type=jnp.float32) + p2b_ref[...]
    posl_ref[...] = jnp.sum(h2 * posh_ref[...], axis=1, keepdims=True)
    negl_ref[...] = jnp.sum(h2 * negh_ref[...], axis=1, keepdims=True)


def _make_tc_mlp(B, L, bb):
    grid = (B // bb,)
    row_spec = lambda w: pl.BlockSpec((bb, w), lambda i: (i, 0))
    rep_spec = lambda r, c: pl.BlockSpec((r, c), lambda i: (0, 0))
    return pl.pallas_call(
        _tc_body,
        grid=grid,
        in_specs=[
            row_spec(L),
            row_spec(E),
            row_spec(E),
            row_spec(E),
            rep_spec(E, E),
            rep_spec(1, E),
            rep_spec(E, 2 * E),
            rep_spec(1, 2 * E),
            rep_spec(E, E),
            rep_spec(1, E),
            rep_spec(E, E),
            rep_spec(1, E),
        ],
        out_specs=[
            row_spec(1),
            row_spec(1),
            row_spec(E),
            row_spec(E),
        ],
        out_shape=[
            jax.ShapeDtypeStruct((B, 1), jnp.float32),
            jax.ShapeDtypeStruct((B, 1), jnp.float32),
            jax.ShapeDtypeStruct((B, E), jnp.float32),
            jax.ShapeDtypeStruct((B, E), jnp.float32),
        ],
    )


def kernel(user, user_hist, user_nbrs, pos_item, neg_item, q_table, p_table,
           q1_w, q1_b, q2_w, q2_b, p1_w, p1_b, p2_w, p2_b):
    B, L = user_hist.shape
    hist = user_hist.astype(jnp.int32)
    pos = pos_item.astype(jnp.int32)
    neg = neg_item.astype(jnp.int32)

    cols = jnp.arange(E, dtype=jnp.int32)[None, :]
    pn_hi = p_table[jnp.concatenate([pos, neg])[:, None], cols]
    pos_hi, neg_hi = pn_hi[:B], pn_hi[B:]

    Ba = (B * 14) // 32
    Bb = B - Ba
    rows_a = jnp.take(q_table, hist[:Ba].reshape(Ba * L), axis=0,
                      mode="clip")
    hu_a = _make_tc_reduce(Ba, L, 16)(rows_a)
    rows_b = jnp.take(q_table, hist[Ba:].reshape(Bb * L), axis=0,
                      mode="clip")
    hu_b = _make_sc_reduce(Bb, L)(rows_b)
    hu_raw = jnp.concatenate([hu_a, hu_b], axis=0)

    posl, negl, mu, logvar = _make_tc_mlp(B, L, 512)(
        hist, hu_raw, pos_hi, neg_hi,
        q1_w, q1_b.reshape(1, E), q2_w, q2_b.reshape(1, 2 * E),
        p1_w, p1_b.reshape(1, E), p2_w, p2_b.reshape(1, E))

    return (posl.reshape(B), negl.reshape(B), mu, logvar)
